# baseline (speedup 1.0000x reference)
import functools

import jax
import jax.numpy as jnp
from jax import lax
from jax.experimental import pallas as pl
from jax.experimental.pallas import tpu as pltpu
from jax.experimental.pallas import tpu_sc as plsc

VOCAB = 1000000
EMB = 64
BATCH = 4096

_NC, _NS = 2, 16
_NW = _NC * _NS
_B_PER_W = BATCH // _NW

@functools.cache
def _make_sc_gather():
    mesh = plsc.VectorSubcoreMesh(core_axis_name="c", subcore_axis_name="s")

    @functools.partial(
        pl.kernel,
        mesh=mesh,
        out_type=[
            jax.ShapeDtypeStruct((BATCH, EMB), jnp.float32),
            jax.ShapeDtypeStruct((BATCH, EMB), jnp.float32),
        ],
        scratch_types=[
            pltpu.VMEM((_B_PER_W,), jnp.int32),
            pltpu.VMEM((_B_PER_W, EMB), jnp.float32),
            pltpu.VMEM((_B_PER_W,), jnp.int32),
            pltpu.VMEM((_B_PER_W, EMB), jnp.float32),
            pltpu.SemaphoreType.DMA,
            pltpu.SemaphoreType.DMA,
        ],
        compiler_params=pltpu.CompilerParams(use_tc_tiling_on_sc=False),
    )
    def _sc_gather(center_hbm, context_hbm, table_hbm, out_c_hbm, out_x_hbm,
                   idx_c, rows_c, idx_x, rows_x, sem_c, sem_x):
        wid = lax.axis_index("s") * _NC + lax.axis_index("c")
        base = wid * _B_PER_W
        pltpu.sync_copy(center_hbm.at[pl.ds(base, _B_PER_W)], idx_c)
        pltpu.sync_copy(context_hbm.at[pl.ds(base, _B_PER_W)], idx_x)
        cp_c = pltpu.async_copy(table_hbm.at[idx_c], rows_c, sem_c)
        cp_x = pltpu.async_copy(table_hbm.at[idx_x], rows_x, sem_x)
        cp_c.wait()
        pltpu.sync_copy(rows_c, out_c_hbm.at[pl.ds(base, _B_PER_W)])
        cp_x.wait()
        pltpu.sync_copy(rows_x, out_x_hbm.at[pl.ds(base, _B_PER_W)])

    return _sc_gather


_BM = 512
_BN = 512


def _mm_body(a_ref, b_ref, o_ref):
    o_ref[...] = lax.dot_general(
        a_ref[...], b_ref[...],
        (((1,), (1,)), ((), ())),
        preferred_element_type=jnp.float32,
    )


def _matmul(center_emb, context_emb):
    return pl.pallas_call(
        _mm_body,
        grid=(BATCH // _BM, BATCH // _BN),
        in_specs=[
            pl.BlockSpec((_BM, EMB), lambda i, j: (i, 0)),
            pl.BlockSpec((_BN, EMB), lambda i, j: (j, 0)),
        ],
        out_specs=pl.BlockSpec((_BM, _BN), lambda i, j: (i, j)),
        out_shape=jax.ShapeDtypeStruct((BATCH, BATCH), jnp.float32),
        compiler_params=pltpu.CompilerParams(
            dimension_semantics=("parallel", "parallel"),
        ),
    )(center_emb, context_emb)


def kernel(center_words, context_words, embeddings):
    center_emb, context_emb = _make_sc_gather()(center_words, context_words, embeddings)
    return _matmul(center_emb, context_emb)


# speedup vs baseline: 1.6191x; 1.6191x over previous
import functools

import jax
import jax.numpy as jnp
from jax import lax
from jax.experimental import pallas as pl
from jax.experimental.pallas import tpu as pltpu
from jax.experimental.pallas import tpu_sc as plsc

VOCAB = 1000000
EMB = 64
BATCH = 4096

_NC, _NS = 2, 16
_NW = _NC * _NS
_B_PER_W = BATCH // _NW


@functools.cache
def _make_sc_gather():
    mesh = plsc.VectorSubcoreMesh(core_axis_name="c", subcore_axis_name="s")

    @functools.partial(
        pl.kernel,
        mesh=mesh,
        out_type=[
            jax.ShapeDtypeStruct((BATCH, EMB), jnp.float32),
            jax.ShapeDtypeStruct((BATCH, EMB), jnp.float32),
        ],
        scratch_types=[
            pltpu.VMEM((_B_PER_W,), jnp.int32),
            pltpu.VMEM((_B_PER_W, EMB), jnp.float32),
            pltpu.SemaphoreType.DMA,
        ],
        compiler_params=pltpu.CompilerParams(needs_layout_passes=False),
    )
    def _sc_gather(center_hbm, context_hbm, table_hbm, out_c_hbm, out_x_hbm,
                   idx_v, out_rows_v, sem):
        wid = lax.axis_index("s") * _NC + lax.axis_index("c")
        base = wid * _B_PER_W

        for words_hbm, out_hbm in ((center_hbm, out_c_hbm),
                                   (context_hbm, out_x_hbm)):
            pltpu.sync_copy(words_hbm.at[pl.ds(base, _B_PER_W)], idx_v)

            def issue(w, _):
                vals = plsc.load_gather(idx_v, [jnp.full((16,), w, jnp.int32)])
                word = lax.reduce_max(vals, (0,))
                pltpu.async_copy(table_hbm.at[word], out_rows_v.at[w], sem)
                return 0

            lax.fori_loop(0, _B_PER_W, issue, 0)

            def drain(w, _):
                pltpu.make_async_copy(
                    table_hbm.at[0], out_rows_v.at[w], sem).wait()
                return 0

            lax.fori_loop(0, _B_PER_W, drain, 0)
            pltpu.sync_copy(out_rows_v, out_hbm.at[pl.ds(base, _B_PER_W)])

    return _sc_gather


_BM = 512
_BN = 512


def _mm_body(a_ref, b_ref, o_ref):
    o_ref[...] = lax.dot_general(
        a_ref[...], b_ref[...],
        (((1,), (1,)), ((), ())),
        preferred_element_type=jnp.float32,
    )


def _matmul(center_emb, context_emb):
    return pl.pallas_call(
        _mm_body,
        grid=(BATCH // _BM, BATCH // _BN),
        in_specs=[
            pl.BlockSpec((_BM, EMB), lambda i, j: (i, 0)),
            pl.BlockSpec((_BN, EMB), lambda i, j: (j, 0)),
        ],
        out_specs=pl.BlockSpec((_BM, _BN), lambda i, j: (i, j)),
        out_shape=jax.ShapeDtypeStruct((BATCH, BATCH), jnp.float32),
        compiler_params=pltpu.CompilerParams(
            dimension_semantics=("parallel", "parallel"),
        ),
    )(center_emb, context_emb)


def kernel(center_words, context_words, embeddings):
    center_emb, context_emb = _make_sc_gather()(
        center_words, context_words, embeddings)
    return _matmul(center_emb, context_emb)


# speedup vs baseline: 1.7338x; 1.0709x over previous
import functools

import jax
import jax.numpy as jnp
from jax import lax
from jax.experimental import pallas as pl
from jax.experimental.pallas import tpu as pltpu
from jax.experimental.pallas import tpu_sc as plsc

VOCAB = 1000000
EMB = 64
BATCH = 4096

_NC, _NS = 2, 16
_NW = _NC * _NS
_B_PER_W = BATCH // _NW


@functools.cache
def _make_sc_gather():
    mesh = plsc.VectorSubcoreMesh(core_axis_name="c", subcore_axis_name="s")

    @functools.partial(
        pl.kernel,
        mesh=mesh,
        out_type=[
            jax.ShapeDtypeStruct((BATCH, EMB), jnp.float32),
            jax.ShapeDtypeStruct((BATCH, EMB), jnp.float32),
        ],
        scratch_types=[
            pltpu.VMEM((_B_PER_W,), jnp.int32),
            pltpu.VMEM((_B_PER_W, EMB), jnp.float32),
            pltpu.SemaphoreType.DMA,
        ],
        compiler_params=pltpu.CompilerParams(needs_layout_passes=False),
    )
    def _sc_gather(center_hbm, context_hbm, table_hbm, out_c_hbm, out_x_hbm,
                   idx_v, out_rows_v, sem):
        wid = lax.axis_index("s") * _NC + lax.axis_index("c")
        base = wid * _B_PER_W

        for words_hbm, out_hbm in ((center_hbm, out_c_hbm),
                                   (context_hbm, out_x_hbm)):
            pltpu.sync_copy(words_hbm.at[pl.ds(base, _B_PER_W)], idx_v)

            def issue(w, _):
                vals = plsc.load_gather(idx_v, [jnp.full((16,), w, jnp.int32)])
                word = lax.reduce_max(vals, (0,))
                pltpu.async_copy(table_hbm.at[word], out_rows_v.at[w], sem)
                return 0

            lax.fori_loop(0, _B_PER_W, issue, 0)

            def drain(w, _):
                pltpu.make_async_copy(
                    table_hbm.at[0], out_rows_v.at[w], sem).wait()
                return 0

            lax.fori_loop(0, _B_PER_W, drain, 0)
            pltpu.sync_copy(out_rows_v, out_hbm.at[pl.ds(base, _B_PER_W)])

    return _sc_gather


_BM = 1024
_BN = 1024


def _mm_body(a_ref, b_ref, o_ref):
    o_ref[...] = lax.dot_general(
        a_ref[...], b_ref[...],
        (((1,), (1,)), ((), ())),
        preferred_element_type=jnp.float32,
    )


def _matmul(center_emb, context_emb):
    return pl.pallas_call(
        _mm_body,
        grid=(BATCH // _BM, BATCH // _BN),
        in_specs=[
            pl.BlockSpec((_BM, EMB), lambda i, j: (i, 0)),
            pl.BlockSpec((_BN, EMB), lambda i, j: (j, 0)),
        ],
        out_specs=pl.BlockSpec((_BM, _BN), lambda i, j: (i, j)),
        out_shape=jax.ShapeDtypeStruct((BATCH, BATCH), jnp.float32),
        compiler_params=pltpu.CompilerParams(
            dimension_semantics=("parallel", "parallel"),
        ),
    )(center_emb, context_emb)


def kernel(center_words, context_words, embeddings):
    center_emb, context_emb = _make_sc_gather()(
        center_words, context_words, embeddings)
    return _matmul(center_emb.astype(jnp.bfloat16),
                   context_emb.astype(jnp.bfloat16))


# speedup vs baseline: 1.7730x; 1.0226x over previous
import functools

import jax
import jax.numpy as jnp
from jax import lax
from jax.experimental import pallas as pl
from jax.experimental.pallas import tpu as pltpu
from jax.experimental.pallas import tpu_sc as plsc

VOCAB = 1000000
EMB = 64
BATCH = 4096

_NC, _NS = 2, 16
_NW = _NC * _NS
_B_PER_W = BATCH // _NW


@functools.cache
def _make_sc_gather():
    mesh = plsc.VectorSubcoreMesh(core_axis_name="c", subcore_axis_name="s")

    @functools.partial(
        pl.kernel,
        mesh=mesh,
        out_type=[
            jax.ShapeDtypeStruct((BATCH, EMB), jnp.float32),
            jax.ShapeDtypeStruct((BATCH, EMB), jnp.float32),
        ],
        scratch_types=[
            pltpu.VMEM((_B_PER_W,), jnp.int32),
            pltpu.VMEM((_B_PER_W, EMB), jnp.float32),
            pltpu.SemaphoreType.DMA,
        ],
        compiler_params=pltpu.CompilerParams(needs_layout_passes=False),
    )
    def _sc_gather(center_hbm, context_hbm, table_hbm, out_c_hbm, out_x_hbm,
                   idx_v, out_rows_v, sem):
        wid = lax.axis_index("s") * _NC + lax.axis_index("c")
        base = wid * _B_PER_W

        for words_hbm, out_hbm in ((center_hbm, out_c_hbm),
                                   (context_hbm, out_x_hbm)):
            pltpu.sync_copy(words_hbm.at[pl.ds(base, _B_PER_W)], idx_v)

            def issue(w, _):
                vals = plsc.load_gather(idx_v, [jnp.full((16,), w, jnp.int32)])
                word = lax.reduce_max(vals, (0,))
                pltpu.async_copy(table_hbm.at[word], out_rows_v.at[w], sem)
                return 0

            lax.fori_loop(0, _B_PER_W, issue, 0)

            def drain(w, _):
                pltpu.make_async_copy(
                    table_hbm.at[0], out_rows_v.at[w], sem).wait()
                return 0

            lax.fori_loop(0, _B_PER_W, drain, 0)
            pltpu.sync_copy(out_rows_v, out_hbm.at[pl.ds(base, _B_PER_W)])

    return _sc_gather


_BM = 512


def _mm_body(a_ref, b_ref, o_ref):
    o_ref[...] = lax.dot_general(
        a_ref[...].astype(jnp.bfloat16), b_ref[...].astype(jnp.bfloat16),
        (((1,), (1,)), ((), ())),
        preferred_element_type=jnp.float32,
    )


def _matmul(center_emb, context_emb):
    return pl.pallas_call(
        _mm_body,
        grid=(BATCH // _BM,),
        in_specs=[
            pl.BlockSpec((_BM, EMB), lambda i: (i, 0)),
            pl.BlockSpec((BATCH, EMB), lambda i: (0, 0)),
        ],
        out_specs=pl.BlockSpec((_BM, BATCH), lambda i: (i, 0)),
        out_shape=jax.ShapeDtypeStruct((BATCH, BATCH), jnp.float32),
        compiler_params=pltpu.CompilerParams(
            dimension_semantics=("arbitrary",),
        ),
    )(center_emb, context_emb)


def kernel(center_words, context_words, embeddings):
    center_emb, context_emb = _make_sc_gather()(
        center_words, context_words, embeddings)
    return _matmul(center_emb, context_emb)
